# baseline (speedup 1.0000x reference)
import functools

import jax
import jax.numpy as jnp
from jax import lax
from jax.experimental import pallas as pl
from jax.experimental.pallas import tpu as pltpu
from jax.experimental.pallas import tpu_sc as plsc

_N = 10000
_E = 160000
_D = 256
_HALF = 128
_B = 64
_T = 32
_C = 10

_NSUB = 16
_CHUNK = 128
_NCH = 2 * (-(-_E // (_NSUB * _CHUNK * 2)))
_HNCH = _NCH // 2
_EPS = _NCH * _CHUNK
_EPAD = _NSUB * _EPS
_RPT = (_N // _NSUB) // 8 * 8
_TAIL = _N - _NSUB * _RPT
_NPAD = _N + 8

_f32 = jnp.float32


def _agg_body(xa, xb, srcp, dstp, oa, ob, sidx, didx, rows0, rows1, acc,
              sem0, sem1):
    c = lax.axis_index("c")
    s = lax.axis_index("s")

    @pl.when(c == 0)
    def _():
        pltpu.sync_copy(xa.at[pl.ds(s * _RPT, _RPT)], acc.at[pl.ds(s * _RPT, _RPT)])

    @pl.when(c == 1)
    def _():
        pltpu.sync_copy(xb.at[pl.ds(s * _RPT, _RPT)], acc.at[pl.ds(s * _RPT, _RPT)])

    base = _NSUB * _RPT
    @pl.when((c == 0) & (s == 0))
    def _():
        pltpu.sync_copy(xa.at[pl.ds(base, _TAIL)], acc.at[pl.ds(base, _TAIL)])

    @pl.when((c == 1) & (s == 0))
    def _():
        pltpu.sync_copy(xb.at[pl.ds(base, _TAIL)], acc.at[pl.ds(base, _TAIL)])

    plsc.subcore_barrier()

    def pipeline(tbl):
        for h in range(2):
            pltpu.sync_copy(srcp.at[s, pl.ds(h * _HNCH, _HNCH)], sidx)
            pltpu.sync_copy(dstp.at[s, pl.ds(h * _HNCH, _HNCH)], didx)
            pltpu.async_copy(tbl.at[sidx.at[0]], rows0, sem0)
            pltpu.async_copy(tbl.at[sidx.at[1]], rows1, sem1)

            def chunk(j, carry):
                for b, (rb, sb) in enumerate(((rows0, sem0), (rows1, sem1))):
                    cj = 2 * j + b
                    pltpu.make_async_copy(tbl.at[sidx.at[0]], rb, sb).wait()
                    pltpu.sync_copy(rb, acc.at[didx.at[cj]], add=True)

                    @pl.when(cj + 2 < _HNCH)
                    def _():
                        pltpu.async_copy(tbl.at[sidx.at[cj + 2]], rb, sb)
                return carry

            lax.fori_loop(0, _HNCH // 2, chunk, 0)

    @pl.when(c == 0)
    def _():
        pipeline(xa)

    @pl.when(c == 1)
    def _():
        pipeline(xb)

    plsc.subcore_barrier()

    @pl.when(c == 0)
    def _():
        pltpu.sync_copy(acc.at[pl.ds(s * _RPT, _RPT)], oa.at[pl.ds(s * _RPT, _RPT)])

    @pl.when(c == 1)
    def _():
        pltpu.sync_copy(acc.at[pl.ds(s * _RPT, _RPT)], ob.at[pl.ds(s * _RPT, _RPT)])

    @pl.when((c == 0) & (s == 0))
    def _():
        pltpu.sync_copy(acc.at[pl.ds(base, _TAIL)], oa.at[pl.ds(base, _TAIL)])

    @pl.when((c == 1) & (s == 0))
    def _():
        pltpu.sync_copy(acc.at[pl.ds(base, _TAIL)], ob.at[pl.ds(base, _TAIL)])


@functools.cache
def _agg_kernel():
    return pl.kernel(
        _agg_body,
        out_type=(
            jax.ShapeDtypeStruct((_N, _HALF), _f32),
            jax.ShapeDtypeStruct((_N, _HALF), _f32),
        ),
        mesh=plsc.VectorSubcoreMesh(
            core_axis_name="c", subcore_axis_name="s", num_cores=2,
            num_subcores=_NSUB),
        scratch_types=[
            pltpu.VMEM((_HNCH, _CHUNK), jnp.int32),
            pltpu.VMEM((_HNCH, _CHUNK), jnp.int32),
            pltpu.VMEM((_CHUNK, _HALF), _f32),
            pltpu.VMEM((_CHUNK, _HALF), _f32),
            pltpu.VMEM_SHARED((_NPAD, _HALF), _f32),
            pltpu.SemaphoreType.DMA,
            pltpu.SemaphoreType.DMA,
        ],
    )


def _agg(xa, xb, srcp, dstp):
    return _agg_kernel()(xa, xb, srcp, dstp)


_ROWS = 1000
_G = _N // _ROWS


def _mlp1_body(pa, pb, w1t, s1, t1, w2t, b2, oa, ob):
    a = jnp.concatenate([pa[...], pb[...]], axis=1)
    h = jnp.dot(a, w1t[...], preferred_element_type=_f32)
    h = jnp.maximum(h * s1[...] + t1[...], 0.0)
    z = jnp.dot(h, w2t[...], preferred_element_type=_f32) + b2[...]
    z = jnp.maximum(z, 0.0)
    oa[...] = z[:, :_HALF]
    ob[...] = z[:, _HALF:]


def _mlp1(pa, pb, w1t, s1, t1, w2t, b2):
    row = pl.BlockSpec((_ROWS, _HALF), lambda i: (i, 0))
    mat = pl.BlockSpec((_D, _D), lambda i: (0, 0))
    vec = pl.BlockSpec((1, _D), lambda i: (0, 0))
    return pl.pallas_call(
        _mlp1_body,
        grid=(_G,),
        in_specs=[row, row, mat, vec, vec, mat, vec],
        out_specs=(row, row),
        out_shape=(
            jax.ShapeDtypeStruct((_N, _HALF), _f32),
            jax.ShapeDtypeStruct((_N, _HALF), _f32),
        ),
    )(pa, pb, w1t, s1, t1, w2t, b2)


def _mlp2pool_body(qa, qb, w3t, s2, t2, bt3, gs):
    i = pl.program_id(0)
    a = jnp.concatenate([qa[...], qb[...]], axis=1)
    h = jnp.dot(a, w3t[...], preferred_element_type=_f32)
    h = jnp.maximum(h * s2[...] + t2[...], 0.0)
    seg = lax.broadcasted_iota(jnp.int32, (_B, _ROWS), 0)
    onehot = (seg == bt3[0]).astype(_f32)
    part = jnp.dot(onehot, h, preferred_element_type=_f32)

    @pl.when(i == 0)
    def _():
        gs[...] = jnp.zeros_like(gs)

    gs[...] += part


def _mlp2pool(qa, qb, w3t, s2, t2, bt3):
    row = pl.BlockSpec((_ROWS, _HALF), lambda i: (i, 0))
    mat = pl.BlockSpec((_D, _D), lambda i: (0, 0))
    vec = pl.BlockSpec((1, _D), lambda i: (0, 0))
    bspec = pl.BlockSpec((1, 1, _ROWS), lambda i: (i, 0, 0))
    return pl.pallas_call(
        _mlp2pool_body,
        grid=(_G,),
        in_specs=[row, row, mat, vec, vec, bspec],
        out_specs=pl.BlockSpec((_B, _D), lambda i: (0, 0)),
        out_shape=jax.ShapeDtypeStruct((_B, _D), _f32),
    )(qa, qb, w3t, s2, t2, bt3)


def _head_body(gs, tv, wtt, bt, wca, wcb, bc, out):
    gt = jnp.dot(tv[...], wtt[...], preferred_element_type=_f32) + bt[...]
    gt = jnp.maximum(gt, 0.0)
    out[...] = (
        jnp.dot(gs[...], wca[...], preferred_element_type=_f32)
        + jnp.dot(gt, wcb[...], preferred_element_type=_f32)
        + bc[...]
    )


def _head(gs, tv, wtt, bt, wca, wcb, bc):
    return pl.pallas_call(
        _head_body,
        out_shape=jax.ShapeDtypeStruct((_B, _C), _f32),
    )(gs, tv, wtt, bt, wca, wcb, bc)


def kernel(x, edge_index, batch, topo_vec, W1, b1, g1, be1, rm1, rv1,
           W2, b2, W3, b3, g2, be2, rm2, rv2, Wt, bt, Wc, bc):
    src = edge_index[0].astype(jnp.int32)
    dst = edge_index[1].astype(jnp.int32)
    pad = _EPAD - _E
    srcp = jnp.concatenate([src, jnp.zeros((pad,), jnp.int32)]).reshape(
        _NSUB, _NCH, _CHUNK)
    dstp = jnp.concatenate([dst, jnp.full((pad,), _N, jnp.int32)]).reshape(
        _NSUB, _NCH, _CHUNK)

    xa = x[:, :_HALF]
    xb = x[:, _HALF:]

    s1 = (g1 * lax.rsqrt(rv1 + 1e-5)).reshape(1, _D)
    t1 = ((b1 - rm1) * s1[0] + be1).reshape(1, _D)
    s2 = (g2 * lax.rsqrt(rv2 + 1e-5)).reshape(1, _D)
    t2 = ((b3 - rm2) * s2[0] + be2).reshape(1, _D)
    b2r = b2.reshape(1, _D)
    btr = bt.reshape(1, _D)
    bcr = bc.reshape(1, _C)
    wct = Wc.T
    bt3 = batch.astype(jnp.int32).reshape(_G, 1, _ROWS)

    pa, pb = _agg(xa, xb, srcp, dstp)
    ha, hb = _mlp1(pa, pb, W1.T, s1, t1, W2.T, b2r)
    qa, qb = _agg(ha, hb, srcp, dstp)
    gs = _mlp2pool(qa, qb, W3.T, s2, t2, bt3)
    return _head(gs, topo_vec, Wt.T, btr, wct[:_D], wct[_D:], bcr)


# speedup vs baseline: 1.1359x; 1.1359x over previous
import functools

import jax
import jax.numpy as jnp
from jax import lax
from jax.experimental import pallas as pl
from jax.experimental.pallas import tpu as pltpu
from jax.experimental.pallas import tpu_sc as plsc

_N = 10000
_E = 160000
_D = 256
_HALF = 128
_B = 64
_T = 32
_C = 10

_NSUB = 16
_CHUNK = 128
_NCH = -(-_E // (_NSUB * _CHUNK))
_EPS = _NCH * _CHUNK
_EPAD = _NSUB * _EPS
_RPT = (_N // _NSUB) // 8 * 8
_TAIL = _N - _NSUB * _RPT
_NPAD = _N + 8

_f32 = jnp.float32


def _agg_body(xa, xb, srcp, dstp, oa, ob, sidx, didx, rows, acc, sem):
    c = lax.axis_index("c")
    s = lax.axis_index("s")

    pltpu.sync_copy(srcp.at[s], sidx)
    pltpu.sync_copy(dstp.at[s], didx)

    @pl.when(c == 0)
    def _():
        pltpu.sync_copy(xa.at[pl.ds(s * _RPT, _RPT)], acc.at[pl.ds(s * _RPT, _RPT)])

    @pl.when(c == 1)
    def _():
        pltpu.sync_copy(xb.at[pl.ds(s * _RPT, _RPT)], acc.at[pl.ds(s * _RPT, _RPT)])

    base = _NSUB * _RPT
    @pl.when((c == 0) & (s == 0))
    def _():
        pltpu.sync_copy(xa.at[pl.ds(base, _TAIL)], acc.at[pl.ds(base, _TAIL)])

    @pl.when((c == 1) & (s == 0))
    def _():
        pltpu.sync_copy(xb.at[pl.ds(base, _TAIL)], acc.at[pl.ds(base, _TAIL)])

    plsc.subcore_barrier()

    def chunk(j, carry):
        @pl.when(c == 0)
        def _():
            pltpu.async_copy(xa.at[sidx.at[j]], rows, sem).wait()

        @pl.when(c == 1)
        def _():
            pltpu.async_copy(xb.at[sidx.at[j]], rows, sem).wait()

        pltpu.sync_copy(rows, acc.at[didx.at[j]], add=True)
        return carry

    lax.fori_loop(0, _NCH, chunk, 0)
    plsc.subcore_barrier()

    @pl.when(c == 0)
    def _():
        pltpu.sync_copy(acc.at[pl.ds(s * _RPT, _RPT)], oa.at[pl.ds(s * _RPT, _RPT)])

    @pl.when(c == 1)
    def _():
        pltpu.sync_copy(acc.at[pl.ds(s * _RPT, _RPT)], ob.at[pl.ds(s * _RPT, _RPT)])

    @pl.when((c == 0) & (s == 0))
    def _():
        pltpu.sync_copy(acc.at[pl.ds(base, _TAIL)], oa.at[pl.ds(base, _TAIL)])

    @pl.when((c == 1) & (s == 0))
    def _():
        pltpu.sync_copy(acc.at[pl.ds(base, _TAIL)], ob.at[pl.ds(base, _TAIL)])


@functools.cache
def _agg_kernel():
    return pl.kernel(
        _agg_body,
        out_type=(
            jax.ShapeDtypeStruct((_N, _HALF), _f32),
            jax.ShapeDtypeStruct((_N, _HALF), _f32),
        ),
        mesh=plsc.VectorSubcoreMesh(
            core_axis_name="c", subcore_axis_name="s", num_cores=2,
            num_subcores=_NSUB),
        scratch_types=[
            pltpu.VMEM((_NCH, _CHUNK), jnp.int32),
            pltpu.VMEM((_NCH, _CHUNK), jnp.int32),
            pltpu.VMEM((_CHUNK, _HALF), _f32),
            pltpu.VMEM_SHARED((_NPAD, _HALF), _f32),
            pltpu.SemaphoreType.DMA,
        ],
    )


def _agg(xa, xb, srcp, dstp):
    return _agg_kernel()(xa, xb, srcp, dstp)


_ROWS = 2000
_G = _N // _ROWS


def _mlp1_body(pa, pb, w1t, s1, t1, w2t, b2, oa, ob):
    a = jnp.concatenate([pa[...], pb[...]], axis=1)
    h = jnp.dot(a, w1t[...], preferred_element_type=_f32)
    h = jnp.maximum(h * s1[...] + t1[...], 0.0)
    z = jnp.dot(h, w2t[...], preferred_element_type=_f32) + b2[...]
    z = jnp.maximum(z, 0.0)
    oa[...] = z[:, :_HALF]
    ob[...] = z[:, _HALF:]


def _mlp1(pa, pb, w1t, s1, t1, w2t, b2):
    row = pl.BlockSpec((_ROWS, _HALF), lambda i: (i, 0))
    mat = pl.BlockSpec((_D, _D), lambda i: (0, 0))
    vec = pl.BlockSpec((1, _D), lambda i: (0, 0))
    return pl.pallas_call(
        _mlp1_body,
        grid=(_G,),
        in_specs=[row, row, mat, vec, vec, mat, vec],
        out_specs=(row, row),
        out_shape=(
            jax.ShapeDtypeStruct((_N, _HALF), _f32),
            jax.ShapeDtypeStruct((_N, _HALF), _f32),
        ),
    )(pa, pb, w1t, s1, t1, w2t, b2)


def _mlp2pool_body(qa, qb, w3t, s2, t2, bt3, tv, wtt, btr, wca, wcb, bcr,
                   out, gs):
    i = pl.program_id(0)
    a = jnp.concatenate([qa[...], qb[...]], axis=1)
    h = jnp.dot(a, w3t[...], preferred_element_type=_f32)
    h = jnp.maximum(h * s2[...] + t2[...], 0.0)
    seg = lax.broadcasted_iota(jnp.int32, (_B, _ROWS), 0)
    onehot = (seg == bt3[0]).astype(_f32)
    part = jnp.dot(onehot, h, preferred_element_type=_f32)

    @pl.when(i == 0)
    def _():
        gs[...] = jnp.zeros_like(gs)

    gs[...] += part

    @pl.when(i == _G - 1)
    def _():
        gt = jnp.dot(tv[...], wtt[...], preferred_element_type=_f32) + btr[...]
        gt = jnp.maximum(gt, 0.0)
        out[...] = (
            jnp.dot(gs[...], wca[...], preferred_element_type=_f32)
            + jnp.dot(gt, wcb[...], preferred_element_type=_f32)
            + bcr[...]
        )


def _fix(*shape):
    return pl.BlockSpec(shape, lambda i, _s=shape: tuple(0 for _ in _s))


def _mlp2pool(qa, qb, w3t, s2, t2, bt3, tv, wtt, btr, wca, wcb, bcr):
    row = pl.BlockSpec((_ROWS, _HALF), lambda i: (i, 0))
    mat = pl.BlockSpec((_D, _D), lambda i: (0, 0))
    vec = pl.BlockSpec((1, _D), lambda i: (0, 0))
    out, _ = pl.pallas_call(
        _mlp2pool_body,
        grid=(_G,),
        in_specs=[row, row, mat, vec, vec,
                  pl.BlockSpec((1, 1, _ROWS), lambda i: (i, 0, 0)),
                  _fix(_B, _T), _fix(_T, _D), _fix(1, _D),
                  _fix(_D, _C), _fix(_D, _C), _fix(1, _C)],
        out_specs=(_fix(_B, _C), _fix(_B, _D)),
        out_shape=(
            jax.ShapeDtypeStruct((_B, _C), _f32),
            jax.ShapeDtypeStruct((_B, _D), _f32),
        ),
    )(qa, qb, w3t, s2, t2, bt3, tv, wtt, btr, wca, wcb, bcr)
    return out


def kernel(x, edge_index, batch, topo_vec, W1, b1, g1, be1, rm1, rv1,
           W2, b2, W3, b3, g2, be2, rm2, rv2, Wt, bt, Wc, bc):
    src = edge_index[0].astype(jnp.int32)
    dst = edge_index[1].astype(jnp.int32)
    pad = _EPAD - _E
    srcp = jnp.concatenate([src, jnp.zeros((pad,), jnp.int32)]).reshape(
        _NSUB, _NCH, _CHUNK)
    dstp = jnp.concatenate([dst, jnp.full((pad,), _N, jnp.int32)]).reshape(
        _NSUB, _NCH, _CHUNK)

    xa = x[:, :_HALF]
    xb = x[:, _HALF:]

    s1 = (g1 * lax.rsqrt(rv1 + 1e-5)).reshape(1, _D)
    t1 = ((b1 - rm1) * s1[0] + be1).reshape(1, _D)
    s2 = (g2 * lax.rsqrt(rv2 + 1e-5)).reshape(1, _D)
    t2 = ((b3 - rm2) * s2[0] + be2).reshape(1, _D)
    b2r = b2.reshape(1, _D)
    btr = bt.reshape(1, _D)
    bcr = bc.reshape(1, _C)
    wct = Wc.T
    bt3 = batch.astype(jnp.int32).reshape(_G, 1, _ROWS)

    pa, pb = _agg(xa, xb, srcp, dstp)
    ha, hb = _mlp1(pa, pb, W1.T, s1, t1, W2.T, b2r)
    qa, qb = _agg(ha, hb, srcp, dstp)
    return _mlp2pool(qa, qb, W3.T, s2, t2, bt3, topo_vec, Wt.T, btr,
                     wct[:_D], wct[_D:], bcr)


# speedup vs baseline: 1.1375x; 1.0014x over previous
import functools

import jax
import jax.numpy as jnp
from jax import lax
from jax.experimental import pallas as pl
from jax.experimental.pallas import tpu as pltpu
from jax.experimental.pallas import tpu_sc as plsc

_N = 10000
_E = 160000
_D = 256
_HALF = 128
_B = 64
_T = 32
_C = 10

_NSUB = 16
_CHUNK = 128
_NCH = -(-_E // (_NSUB * _CHUNK))
_EPS = _NCH * _CHUNK
_EPAD = _NSUB * _EPS
_RPT = (_N // _NSUB) // 8 * 8
_TAIL = _N - _NSUB * _RPT
_NPAD = _N + 8

_f32 = jnp.float32


def _agg_body(xa, xb, srcp, dstp, oa, ob, sidx, didx, rows, acc, sem):
    c = lax.axis_index("c")
    s = lax.axis_index("s")

    pltpu.sync_copy(srcp.at[s], sidx)
    pltpu.sync_copy(dstp.at[s], didx)

    @pl.when(c == 0)
    def _():
        pltpu.sync_copy(xa.at[pl.ds(s * _RPT, _RPT)], acc.at[pl.ds(s * _RPT, _RPT)])

    @pl.when(c == 1)
    def _():
        pltpu.sync_copy(xb.at[pl.ds(s * _RPT, _RPT)], acc.at[pl.ds(s * _RPT, _RPT)])

    base = _NSUB * _RPT
    @pl.when((c == 0) & (s == 0))
    def _():
        pltpu.sync_copy(xa.at[pl.ds(base, _TAIL)], acc.at[pl.ds(base, _TAIL)])

    @pl.when((c == 1) & (s == 0))
    def _():
        pltpu.sync_copy(xb.at[pl.ds(base, _TAIL)], acc.at[pl.ds(base, _TAIL)])

    plsc.subcore_barrier()

    def chunk(j, carry):
        @pl.when(c == 0)
        def _():
            pltpu.async_copy(xa.at[sidx.at[j]], rows, sem).wait()

        @pl.when(c == 1)
        def _():
            pltpu.async_copy(xb.at[sidx.at[j]], rows, sem).wait()

        pltpu.sync_copy(rows, acc.at[didx.at[j]], add=True)
        return carry

    lax.fori_loop(0, _NCH, chunk, 0)
    plsc.subcore_barrier()

    @pl.when(c == 0)
    def _():
        pltpu.sync_copy(acc.at[pl.ds(s * _RPT, _RPT)], oa.at[pl.ds(s * _RPT, _RPT)])

    @pl.when(c == 1)
    def _():
        pltpu.sync_copy(acc.at[pl.ds(s * _RPT, _RPT)], ob.at[pl.ds(s * _RPT, _RPT)])

    @pl.when((c == 0) & (s == 0))
    def _():
        pltpu.sync_copy(acc.at[pl.ds(base, _TAIL)], oa.at[pl.ds(base, _TAIL)])

    @pl.when((c == 1) & (s == 0))
    def _():
        pltpu.sync_copy(acc.at[pl.ds(base, _TAIL)], ob.at[pl.ds(base, _TAIL)])


@functools.cache
def _agg_kernel():
    return pl.kernel(
        _agg_body,
        out_type=(
            jax.ShapeDtypeStruct((_N, _HALF), _f32),
            jax.ShapeDtypeStruct((_N, _HALF), _f32),
        ),
        mesh=plsc.VectorSubcoreMesh(
            core_axis_name="c", subcore_axis_name="s", num_cores=2,
            num_subcores=_NSUB),
        scratch_types=[
            pltpu.VMEM((_NCH, _CHUNK), jnp.int32),
            pltpu.VMEM((_NCH, _CHUNK), jnp.int32),
            pltpu.VMEM((_CHUNK, _HALF), _f32),
            pltpu.VMEM_SHARED((_NPAD, _HALF), _f32),
            pltpu.SemaphoreType.DMA,
        ],
        compiler_params=pltpu.CompilerParams(use_tc_tiling_on_sc=True),
    )


def _agg(xa, xb, srcp, dstp):
    return _agg_kernel()(xa, xb, srcp, dstp)


_ROWS = 2000
_G = _N // _ROWS


def _mlp1_body(pa, pb, w1t, s1, t1, w2t, b2, oa, ob):
    a = jnp.concatenate([pa[...], pb[...]], axis=1)
    h = jnp.dot(a, w1t[...], preferred_element_type=_f32)
    h = jnp.maximum(h * s1[...] + t1[...], 0.0)
    z = jnp.dot(h, w2t[...], preferred_element_type=_f32) + b2[...]
    z = jnp.maximum(z, 0.0)
    oa[...] = z[:, :_HALF]
    ob[...] = z[:, _HALF:]


def _mlp1(pa, pb, w1t, s1, t1, w2t, b2):
    row = pl.BlockSpec((_ROWS, _HALF), lambda i: (i, 0))
    mat = pl.BlockSpec((_D, _D), lambda i: (0, 0))
    vec = pl.BlockSpec((1, _D), lambda i: (0, 0))
    return pl.pallas_call(
        _mlp1_body,
        grid=(_G,),
        in_specs=[row, row, mat, vec, vec, mat, vec],
        out_specs=(row, row),
        out_shape=(
            jax.ShapeDtypeStruct((_N, _HALF), _f32),
            jax.ShapeDtypeStruct((_N, _HALF), _f32),
        ),
    )(pa, pb, w1t, s1, t1, w2t, b2)


def _mlp2pool_body(qa, qb, w3t, s2, t2, bt3, tv, wtt, btr, wca, wcb, bcr,
                   out, gs):
    i = pl.program_id(0)
    a = jnp.concatenate([qa[...], qb[...]], axis=1)
    h = jnp.dot(a, w3t[...], preferred_element_type=_f32)
    h = jnp.maximum(h * s2[...] + t2[...], 0.0)
    seg = lax.broadcasted_iota(jnp.int32, (_B, _ROWS), 0)
    onehot = (seg == bt3[0]).astype(_f32)
    part = jnp.dot(onehot, h, preferred_element_type=_f32)

    @pl.when(i == 0)
    def _():
        gs[...] = jnp.zeros_like(gs)

    gs[...] += part

    @pl.when(i == _G - 1)
    def _():
        gt = jnp.dot(tv[...], wtt[...], preferred_element_type=_f32) + btr[...]
        gt = jnp.maximum(gt, 0.0)
        out[...] = (
            jnp.dot(gs[...], wca[...], preferred_element_type=_f32)
            + jnp.dot(gt, wcb[...], preferred_element_type=_f32)
            + bcr[...]
        )


def _fix(*shape):
    return pl.BlockSpec(shape, lambda i, _s=shape: tuple(0 for _ in _s))


def _mlp2pool(qa, qb, w3t, s2, t2, bt3, tv, wtt, btr, wca, wcb, bcr):
    row = pl.BlockSpec((_ROWS, _HALF), lambda i: (i, 0))
    mat = pl.BlockSpec((_D, _D), lambda i: (0, 0))
    vec = pl.BlockSpec((1, _D), lambda i: (0, 0))
    out, _ = pl.pallas_call(
        _mlp2pool_body,
        grid=(_G,),
        in_specs=[row, row, mat, vec, vec,
                  pl.BlockSpec((1, 1, _ROWS), lambda i: (i, 0, 0)),
                  _fix(_B, _T), _fix(_T, _D), _fix(1, _D),
                  _fix(_D, _C), _fix(_D, _C), _fix(1, _C)],
        out_specs=(_fix(_B, _C), _fix(_B, _D)),
        out_shape=(
            jax.ShapeDtypeStruct((_B, _C), _f32),
            jax.ShapeDtypeStruct((_B, _D), _f32),
        ),
    )(qa, qb, w3t, s2, t2, bt3, tv, wtt, btr, wca, wcb, bcr)
    return out


def kernel(x, edge_index, batch, topo_vec, W1, b1, g1, be1, rm1, rv1,
           W2, b2, W3, b3, g2, be2, rm2, rv2, Wt, bt, Wc, bc):
    src = edge_index[0].astype(jnp.int32)
    dst = edge_index[1].astype(jnp.int32)
    pad = _EPAD - _E
    srcp = jnp.concatenate([src, jnp.zeros((pad,), jnp.int32)]).reshape(
        _NSUB, _NCH, _CHUNK)
    dstp = jnp.concatenate([dst, jnp.full((pad,), _N, jnp.int32)]).reshape(
        _NSUB, _NCH, _CHUNK)

    xa = x[:, :_HALF]
    xb = x[:, _HALF:]

    s1 = (g1 * lax.rsqrt(rv1 + 1e-5)).reshape(1, _D)
    t1 = ((b1 - rm1) * s1[0] + be1).reshape(1, _D)
    s2 = (g2 * lax.rsqrt(rv2 + 1e-5)).reshape(1, _D)
    t2 = ((b3 - rm2) * s2[0] + be2).reshape(1, _D)
    b2r = b2.reshape(1, _D)
    btr = bt.reshape(1, _D)
    bcr = bc.reshape(1, _C)
    wct = Wc.T
    bt3 = batch.astype(jnp.int32).reshape(_G, 1, _ROWS)

    pa, pb = _agg(xa, xb, srcp, dstp)
    ha, hb = _mlp1(pa, pb, W1.T, s1, t1, W2.T, b2r)
    qa, qb = _agg(ha, hb, srcp, dstp)
    return _mlp2pool(qa, qb, W3.T, s2, t2, bt3, topo_vec, Wt.T, btr,
                     wct[:_D], wct[_D:], bcr)
